# baseline (speedup 1.0000x reference)
import functools

import jax
import jax.numpy as jnp
from jax import lax
from jax.experimental import pallas as pl
from jax.experimental.pallas import tpu as pltpu
from jax.experimental.pallas import tpu_sc as plsc

N = 10000
E = 160000
D_IN = 640
D_HID = 64

NW = 32
EW = E // NW
CH = 125
NCH = EW // CH
NP = 10240
RPT = NP // 16
ZCH = 128

def _zero_vmem(buf, rows, width):
    def body(r, carry):
        for c4 in range(width // 16):
            buf[r, pl.ds(c4 * 16, 16)] = jnp.zeros((16,), jnp.float32)
        return carry
    lax.fori_loop(0, rows, body, 0)


@functools.cache
def _build_sc_degree():
  mesh = plsc.VectorSubcoreMesh(core_axis_name="c", subcore_axis_name="s")

  @functools.partial(
      pl.kernel,
      mesh=mesh,
      out_type=jax.ShapeDtypeStruct((NW, N), jnp.float32),
      scratch_types=[
          pltpu.VMEM((EW,), jnp.int32),
          pltpu.VMEM((N,), jnp.float32),
      ],
      compiler_params=pltpu.CompilerParams(use_tc_tiling_on_sc=False,
                                           needs_layout_passes=False),
  )
  def _sc_degree(dst_hbm, out_hbm, dst_v, deg_v):
    cid = lax.axis_index("c")
    sid = lax.axis_index("s")
    wid = sid * 2 + cid

    def zb(i, carry):
        deg_v[pl.ds(i * 16, 16)] = jnp.zeros((16,), jnp.float32)
        return carry
    lax.fori_loop(0, N // 16, zb, 0)

    pltpu.sync_copy(dst_hbm.at[wid], dst_v)
    ones = jnp.ones((16,), jnp.float32)

    def body(i, carry):
        idx = dst_v[pl.ds(i * 16, 16)]
        plsc.addupdate_scatter(deg_v, [idx], ones)
        return carry
    lax.fori_loop(0, (EW // 16) * 16 // 16, body, 0)
    idx_t = dst_v[pl.ds(EW - 16, 16)]
    lanemask = lax.iota(jnp.int32, 16) >= (16 - (EW - (EW // 16) * 16) or 16)
    plsc.addupdate_scatter(deg_v, [idx_t], ones, mask=lanemask)

    pltpu.sync_copy(deg_v, out_hbm.at[wid])

  return _sc_degree


@functools.cache
def _build_sc_propagate(width, cache_u):
  mesh = plsc.VectorSubcoreMesh(core_axis_name="c", subcore_axis_name="s")

  scratch = [
      pltpu.VMEM((NCH, CH), jnp.int32),
      pltpu.VMEM((NCH, CH), jnp.int32),
      pltpu.VMEM((CH, width), jnp.float32),
      pltpu.VMEM((CH, width), jnp.float32),
      pltpu.VMEM_SHARED((NP, width), jnp.float32),
  ]
  if cache_u:
      scratch.append(pltpu.VMEM_SHARED((N, width), jnp.float32))
  scratch += [pltpu.SemaphoreType.DMA, pltpu.SemaphoreType.DMA]

  @functools.partial(
      pl.kernel,
      mesh=mesh,
      out_type=jax.ShapeDtypeStruct((2, NP, width), jnp.float32),
      scratch_types=scratch,
      compiler_params=pltpu.CompilerParams(use_tc_tiling_on_sc=False),
  )
  def _sc_propagate(src_hbm, dst_hbm, u_hbm, out_hbm,
                    src_v, dst_v, rows_a, rows_b, acc_sh, *rest):
    if cache_u:
        u_sh, sem_a, sem_b = rest
    else:
        sem_a, sem_b = rest
    cid = lax.axis_index("c")
    sid = lax.axis_index("s")
    wid = sid * 2 + cid

    if cache_u:
        pltpu.sync_copy(u_hbm.at[pl.ds(sid * 640, 400)],
                        u_sh.at[pl.ds(sid * 640, 400)])
        @pl.when(sid < 15)
        def _():
            pltpu.sync_copy(u_hbm.at[pl.ds(sid * 640 + 400, 240)],
                            u_sh.at[pl.ds(sid * 640 + 400, 240)])

    _zero_vmem(rows_a, CH, width)
    for j in range(RPT // 120):
        pltpu.sync_copy(rows_a.at[pl.ds(0, 120)],
                        acc_sh.at[pl.ds(sid * RPT + j * 120, 120)])
    pltpu.sync_copy(rows_a.at[pl.ds(0, RPT - (RPT // 120) * 120)],
                    acc_sh.at[pl.ds(sid * RPT + (RPT // 120) * 120,
                                    RPT - (RPT // 120) * 120)])
    plsc.subcore_barrier()

    pltpu.sync_copy(src_hbm.at[wid], src_v)
    pltpu.sync_copy(dst_hbm.at[wid], dst_v)
    gsrc = u_sh if cache_u else u_hbm
    bufs = (rows_a, rows_b)
    sems = (sem_a, sem_b)
    pend = pltpu.async_copy(gsrc.at[src_v.at[0]], bufs[0], sems[0])
    for c in range(NCH):
        if c + 1 < NCH:
            nxt = pltpu.async_copy(gsrc.at[src_v.at[c + 1]],
                                   bufs[(c + 1) % 2], sems[(c + 1) % 2])
        pend.wait()
        pltpu.sync_copy(bufs[c % 2], acc_sh.at[dst_v.at[c]], add=True)
        if c + 1 < NCH:
            pend = nxt
    plsc.subcore_barrier()

    pltpu.sync_copy(acc_sh.at[pl.ds(sid * RPT, RPT)],
                    out_hbm.at[cid, pl.ds(sid * RPT, RPT)])

  return _sc_propagate


def _k_mm(x_ref, w_ref, o_ref):
    o_ref[...] = jnp.dot(x_ref[...], w_ref[...],
                         preferred_element_type=jnp.float32)


def _k_mm_dinv_u(x_ref, w_ref, dp_ref, dinv_ref, u_ref):
    h = jnp.dot(x_ref[...], w_ref[...], preferred_element_type=jnp.float32)
    deg = 1.0 + lax.dot_general(
        dp_ref[...], jnp.ones((NW, 16), jnp.float32),
        dimension_numbers=(((0,), (0,)), ((), ())),
        preferred_element_type=jnp.float32)
    dinv = lax.rsqrt(deg)
    dinv_ref[...] = dinv
    u_ref[...] = dinv[:, 0:1] * h


def _k_relu_scale(vp_ref, u_ref, dinv_ref, b_ref, ub_ref):
    d = dinv_ref[:, 0:1]
    p = d * (vp_ref[0] + vp_ref[1] + u_ref[...])
    ub_ref[...] = d * jnp.maximum(p + b_ref[...], 0.0)


def _k_dual(vp_ref, u_ref, dinv_ref, ws_ref, bs_ref, ww_ref, bw_ref,
            u34_ref):
    d = dinv_ref[:, 0:1]
    pb = d * (vp_ref[0] + vp_ref[1] + u_ref[...])
    u34_ref[:, 0:D_HID] = d * jnp.maximum(
        jnp.dot(pb, ws_ref[...], preferred_element_type=jnp.float32)
        + bs_ref[...], 0.0)
    u34_ref[:, D_HID:2 * D_HID] = d * jnp.maximum(
        jnp.dot(pb, ww_ref[...], preferred_element_type=jnp.float32)
        + bw_ref[...], 0.0)


def _k_final2(vp_ref, u_ref, dinv_ref, ws_ref, bs_ref, ww_ref, bw_ref,
              xs_ref, xw_ref):
    d = dinv_ref[:, 0:1]
    ps = d * (vp_ref[0, :, 0:D_HID] + vp_ref[1, :, 0:D_HID]
              + u_ref[:, 0:D_HID])
    pw = d * (vp_ref[0, :, D_HID:2 * D_HID] + vp_ref[1, :, D_HID:2 * D_HID]
              + u_ref[:, D_HID:2 * D_HID])
    xs_ref[...] = jnp.maximum(
        jnp.dot(ps, ws_ref[...], preferred_element_type=jnp.float32)
        + bs_ref[...], 0.0)
    xw_ref[...] = jnp.maximum(
        jnp.dot(pw, ww_ref[...], preferred_element_type=jnp.float32)
        + bw_ref[...], 0.0)


def _row_spec(blk, width):
    return pl.BlockSpec((blk, width), lambda i: (i, 0))


def _rep_spec(shape):
    nd = len(shape)
    return pl.BlockSpec(shape, lambda i: (0,) * nd)


def _part_spec(blk, width):
    return pl.BlockSpec((2, blk, width), lambda i: (0, i, 0))


def kernel(x, edges, W_b, b_b, W_s1, b_s1, W_s2, b_s2, W_w1, b_w1, W_w2, b_w2):
    edges = edges.astype(jnp.int32)
    src3 = edges[:, 0].reshape(NW, NCH, CH)
    dst3 = edges[:, 1].reshape(NW, NCH, CH)
    dst2 = edges[:, 1].reshape(NW, EW)
    b_b2 = b_b.reshape(1, D_HID)
    b_s12 = b_s1.reshape(1, D_HID)
    b_s22 = b_s2.reshape(1, D_IN)
    b_w12 = b_w1.reshape(1, D_HID)
    b_w22 = b_w2.reshape(1, D_IN)

    _sc_degree = _build_sc_degree()
    _sc_prop64 = _build_sc_propagate(D_HID, True)
    _sc_prop128 = _build_sc_propagate(2 * D_HID, False)

    degp = _sc_degree(dst2)
    dinv, u1 = pl.pallas_call(
        _k_mm_dinv_u,
        grid=(10,),
        in_specs=[_row_spec(1024, D_IN), _rep_spec((D_IN, D_HID)),
                  pl.BlockSpec((NW, 1024), lambda i: (0, i))],
        out_specs=[_row_spec(1024, 16), _row_spec(1024, D_HID)],
        out_shape=[jax.ShapeDtypeStruct((N, 16), jnp.float32),
                   jax.ShapeDtypeStruct((N, D_HID), jnp.float32)],
    )(x, W_b, degp)

    v1p = _sc_prop64(src3, dst3, u1)

    ub = pl.pallas_call(
        _k_relu_scale,
        grid=(5,),
        in_specs=[_part_spec(2000, D_HID), _row_spec(2000, D_HID),
                  _row_spec(2000, 16), _rep_spec((1, D_HID))],
        out_specs=_row_spec(2000, D_HID),
        out_shape=jax.ShapeDtypeStruct((N, D_HID), jnp.float32),
    )(v1p, u1, dinv, b_b2)

    vbp = _sc_prop64(src3, dst3, ub)

    u34 = pl.pallas_call(
        _k_dual,
        grid=(5,),
        in_specs=[_part_spec(2000, D_HID), _row_spec(2000, D_HID),
                  _row_spec(2000, 16),
                  _rep_spec((D_HID, D_HID)), _rep_spec((1, D_HID)),
                  _rep_spec((D_HID, D_HID)), _rep_spec((1, D_HID))],
        out_specs=_row_spec(2000, 2 * D_HID),
        out_shape=jax.ShapeDtypeStruct((N, 2 * D_HID), jnp.float32),
    )(vbp, ub, dinv, W_s1, b_s12, W_w1, b_w12)

    v34p = _sc_prop128(src3, dst3, u34)
    xs, xw = pl.pallas_call(
        _k_final2,
        grid=(10,),
        in_specs=[_part_spec(1000, 2 * D_HID), _row_spec(1000, 2 * D_HID),
                  _row_spec(1000, 16),
                  _rep_spec((D_HID, D_IN)), _rep_spec((1, D_IN)),
                  _rep_spec((D_HID, D_IN)), _rep_spec((1, D_IN))],
        out_specs=[_row_spec(1000, D_IN), _row_spec(1000, D_IN)],
        out_shape=[jax.ShapeDtypeStruct((N, D_IN), jnp.float32),
                   jax.ShapeDtypeStruct((N, D_IN), jnp.float32)],
    )(v34p, u34, dinv, W_s2, b_s22, W_w2, b_w22)

    return (xs, xw)


# speedup vs baseline: 1.0618x; 1.0618x over previous
import functools

import jax
import jax.numpy as jnp
from jax import lax
from jax.experimental import pallas as pl
from jax.experimental.pallas import tpu as pltpu
from jax.experimental.pallas import tpu_sc as plsc

N = 10000
E = 160000
D_IN = 640
D_HID = 64

NW = 32
EW = E // NW
CH = 125
NCH = EW // CH
NP = 10240
RPT = NP // 16
ZCH = 128

def _zero_vmem(buf, rows, width):
    def body(r, carry):
        for c4 in range(width // 16):
            buf[r, pl.ds(c4 * 16, 16)] = jnp.zeros((16,), jnp.float32)
        return carry
    lax.fori_loop(0, rows, body, 0)


@functools.cache
def _build_sc_degree():
  mesh = plsc.VectorSubcoreMesh(core_axis_name="c", subcore_axis_name="s")

  @functools.partial(
      pl.kernel,
      mesh=mesh,
      out_type=jax.ShapeDtypeStruct((NW, N), jnp.float32),
      scratch_types=[
          pltpu.VMEM((EW,), jnp.int32),
          pltpu.VMEM((N,), jnp.float32),
      ],
      compiler_params=pltpu.CompilerParams(use_tc_tiling_on_sc=False,
                                           needs_layout_passes=False),
  )
  def _sc_degree(dst_hbm, out_hbm, dst_v, deg_v):
    cid = lax.axis_index("c")
    sid = lax.axis_index("s")
    wid = sid * 2 + cid

    def zb(i, carry):
        deg_v[pl.ds(i * 16, 16)] = jnp.zeros((16,), jnp.float32)
        return carry
    lax.fori_loop(0, N // 16, zb, 0)

    pltpu.sync_copy(dst_hbm.at[wid], dst_v)
    ones = jnp.ones((16,), jnp.float32)

    def body(i, carry):
        idx = dst_v[pl.ds(i * 16, 16)]
        plsc.addupdate_scatter(deg_v, [idx], ones)
        return carry
    lax.fori_loop(0, (EW // 16) * 16 // 16, body, 0)
    idx_t = dst_v[pl.ds(EW - 16, 16)]
    lanemask = lax.iota(jnp.int32, 16) >= (16 - (EW - (EW // 16) * 16) or 16)
    plsc.addupdate_scatter(deg_v, [idx_t], ones, mask=lanemask)

    pltpu.sync_copy(deg_v, out_hbm.at[wid])

  return _sc_degree


@functools.cache
def _build_sc_propagate(width, cache_u):
  mesh = plsc.VectorSubcoreMesh(core_axis_name="c", subcore_axis_name="s")

  scratch = [
      pltpu.VMEM((NCH, CH), jnp.int32),
      pltpu.VMEM((NCH, CH), jnp.int32),
      pltpu.VMEM((CH, width), jnp.float32),
      pltpu.VMEM((CH, width), jnp.float32),
      pltpu.VMEM_SHARED((NP, width), jnp.float32),
  ]
  if cache_u:
      scratch.append(pltpu.VMEM_SHARED((N, width), jnp.float32))
  scratch += [pltpu.SemaphoreType.DMA, pltpu.SemaphoreType.DMA]

  @functools.partial(
      pl.kernel,
      mesh=mesh,
      out_type=jax.ShapeDtypeStruct((2, NP, width), jnp.float32),
      scratch_types=scratch,
      compiler_params=pltpu.CompilerParams(use_tc_tiling_on_sc=False),
  )
  def _sc_propagate(src_hbm, dst_hbm, u_hbm, out_hbm,
                    src_v, dst_v, rows_a, rows_b, acc_sh, *rest):
    if cache_u:
        u_sh, sem_a, sem_b = rest
    else:
        sem_a, sem_b = rest
    cid = lax.axis_index("c")
    sid = lax.axis_index("s")
    wid = sid * 2 + cid

    if cache_u:
        pltpu.sync_copy(u_hbm.at[pl.ds(sid * 640, 400)],
                        u_sh.at[pl.ds(sid * 640, 400)])
        @pl.when(sid < 15)
        def _():
            pltpu.sync_copy(u_hbm.at[pl.ds(sid * 640 + 400, 240)],
                            u_sh.at[pl.ds(sid * 640 + 400, 240)])

    _zero_vmem(rows_a, CH, width)
    for j in range(RPT // 120):
        pltpu.sync_copy(rows_a.at[pl.ds(0, 120)],
                        acc_sh.at[pl.ds(sid * RPT + j * 120, 120)])
    pltpu.sync_copy(rows_a.at[pl.ds(0, RPT - (RPT // 120) * 120)],
                    acc_sh.at[pl.ds(sid * RPT + (RPT // 120) * 120,
                                    RPT - (RPT // 120) * 120)])
    plsc.subcore_barrier()

    pltpu.sync_copy(src_hbm.at[wid], src_v)
    pltpu.sync_copy(dst_hbm.at[wid], dst_v)
    gsrc = u_sh if cache_u else u_hbm
    bufs = (rows_a, rows_b)
    sems = (sem_a, sem_b)
    pend = pltpu.async_copy(gsrc.at[src_v.at[0]], bufs[0], sems[0])
    for c in range(NCH):
        if c + 1 < NCH:
            nxt = pltpu.async_copy(gsrc.at[src_v.at[c + 1]],
                                   bufs[(c + 1) % 2], sems[(c + 1) % 2])
        pend.wait()
        pltpu.sync_copy(bufs[c % 2], acc_sh.at[dst_v.at[c]], add=True)
        if c + 1 < NCH:
            pend = nxt
    plsc.subcore_barrier()

    pltpu.sync_copy(acc_sh.at[pl.ds(sid * RPT, RPT)],
                    out_hbm.at[cid, pl.ds(sid * RPT, RPT)])

  return _sc_propagate


def _k_mm(x_ref, w_ref, o_ref):
    o_ref[...] = jnp.dot(x_ref[...], w_ref[...],
                         preferred_element_type=jnp.float32)


def _k_mm_dinv_u(x_ref, w_ref, dp_ref, dinv_ref, u_ref):
    h = jnp.dot(x_ref[...], w_ref[...], preferred_element_type=jnp.float32)
    deg = 1.0 + lax.dot_general(
        dp_ref[...], jnp.ones((NW, 16), jnp.float32),
        dimension_numbers=(((0,), (0,)), ((), ())),
        preferred_element_type=jnp.float32)
    dinv = lax.rsqrt(deg)
    dinv_ref[...] = dinv
    u_ref[...] = dinv[:, 0:1] * h


def _k_relu_scale(vp_ref, u_ref, dinv_ref, b_ref, ub_ref):
    d = dinv_ref[:, 0:1]
    p = d * (vp_ref[0] + vp_ref[1] + u_ref[...])
    ub_ref[...] = d * jnp.maximum(p + b_ref[...], 0.0)


def _k_dual(vp_ref, u_ref, dinv_ref, ws_ref, bs_ref, ww_ref, bw_ref,
            u34_ref):
    d = dinv_ref[:, 0:1]
    pb = d * (vp_ref[0] + vp_ref[1] + u_ref[...])
    u34_ref[:, 0:D_HID] = d * jnp.maximum(
        jnp.dot(pb, ws_ref[...], preferred_element_type=jnp.float32)
        + bs_ref[...], 0.0)
    u34_ref[:, D_HID:2 * D_HID] = d * jnp.maximum(
        jnp.dot(pb, ww_ref[...], preferred_element_type=jnp.float32)
        + bw_ref[...], 0.0)


def _k_final2(vp_ref, u_ref, dinv_ref, ws_ref, bs_ref, ww_ref, bw_ref,
              xs_ref, xw_ref):
    d = dinv_ref[:, 0:1]
    ps = d * (vp_ref[0, :, 0:D_HID] + vp_ref[1, :, 0:D_HID]
              + u_ref[:, 0:D_HID])
    pw = d * (vp_ref[0, :, D_HID:2 * D_HID] + vp_ref[1, :, D_HID:2 * D_HID]
              + u_ref[:, D_HID:2 * D_HID])
    xs_ref[...] = jnp.maximum(
        jnp.dot(ps, ws_ref[...], preferred_element_type=jnp.float32)
        + bs_ref[...], 0.0)
    xw_ref[...] = jnp.maximum(
        jnp.dot(pw, ww_ref[...], preferred_element_type=jnp.float32)
        + bw_ref[...], 0.0)


def _row_spec(blk, width):
    return pl.BlockSpec((blk, width), lambda i: (i, 0))


def _rep_spec(shape):
    nd = len(shape)
    return pl.BlockSpec(shape, lambda i: (0,) * nd)


def _part_spec(blk, width):
    return pl.BlockSpec((2, blk, width), lambda i: (0, i, 0))


def kernel(x, edges, W_b, b_b, W_s1, b_s1, W_s2, b_s2, W_w1, b_w1, W_w2, b_w2):
    edges = edges.astype(jnp.int32)
    src3 = edges[:, 0].reshape(NW, NCH, CH)
    dst3 = edges[:, 1].reshape(NW, NCH, CH)
    dst2 = edges[:, 1].reshape(NW, EW)
    b_b2 = b_b.reshape(1, D_HID)
    b_s12 = b_s1.reshape(1, D_HID)
    b_s22 = b_s2.reshape(1, D_IN)
    b_w12 = b_w1.reshape(1, D_HID)
    b_w22 = b_w2.reshape(1, D_IN)

    _sc_degree = _build_sc_degree()
    _sc_prop64 = _build_sc_propagate(D_HID, False)
    _sc_prop128 = _build_sc_propagate(2 * D_HID, False)

    degp = _sc_degree(dst2)
    dinv, u1 = pl.pallas_call(
        _k_mm_dinv_u,
        grid=(10,),
        in_specs=[_row_spec(1024, D_IN), _rep_spec((D_IN, D_HID)),
                  pl.BlockSpec((NW, 1024), lambda i: (0, i))],
        out_specs=[_row_spec(1024, 16), _row_spec(1024, D_HID)],
        out_shape=[jax.ShapeDtypeStruct((N, 16), jnp.float32),
                   jax.ShapeDtypeStruct((N, D_HID), jnp.float32)],
    )(x, W_b, degp)

    v1p = _sc_prop64(src3, dst3, u1)

    ub = pl.pallas_call(
        _k_relu_scale,
        grid=(5,),
        in_specs=[_part_spec(2000, D_HID), _row_spec(2000, D_HID),
                  _row_spec(2000, 16), _rep_spec((1, D_HID))],
        out_specs=_row_spec(2000, D_HID),
        out_shape=jax.ShapeDtypeStruct((N, D_HID), jnp.float32),
    )(v1p, u1, dinv, b_b2)

    vbp = _sc_prop64(src3, dst3, ub)

    u34 = pl.pallas_call(
        _k_dual,
        grid=(5,),
        in_specs=[_part_spec(2000, D_HID), _row_spec(2000, D_HID),
                  _row_spec(2000, 16),
                  _rep_spec((D_HID, D_HID)), _rep_spec((1, D_HID)),
                  _rep_spec((D_HID, D_HID)), _rep_spec((1, D_HID))],
        out_specs=_row_spec(2000, 2 * D_HID),
        out_shape=jax.ShapeDtypeStruct((N, 2 * D_HID), jnp.float32),
    )(vbp, ub, dinv, W_s1, b_s12, W_w1, b_w12)

    v34p = _sc_prop128(src3, dst3, u34)
    xs, xw = pl.pallas_call(
        _k_final2,
        grid=(10,),
        in_specs=[_part_spec(1000, 2 * D_HID), _row_spec(1000, 2 * D_HID),
                  _row_spec(1000, 16),
                  _rep_spec((D_HID, D_IN)), _rep_spec((1, D_IN)),
                  _rep_spec((D_HID, D_IN)), _rep_spec((1, D_IN))],
        out_specs=[_row_spec(1000, D_IN), _row_spec(1000, D_IN)],
        out_shape=[jax.ShapeDtypeStruct((N, D_IN), jnp.float32),
                   jax.ShapeDtypeStruct((N, D_IN), jnp.float32)],
    )(v34p, u34, dinv, W_s2, b_s22, W_w2, b_w22)

    return (xs, xw)


# speedup vs baseline: 1.1466x; 1.0798x over previous
import functools

import jax
import jax.numpy as jnp
from jax import lax
from jax.experimental import pallas as pl
from jax.experimental.pallas import tpu as pltpu
from jax.experimental.pallas import tpu_sc as plsc

N = 10000
E = 160000
D_IN = 640
D_HID = 64

NW = 32
EW = E // NW
CH = 125
NCH = EW // CH
NBUF = 3
NP = 10240
RPT = NP // 16
ZCH = 128

def _zero_vmem(buf, rows, width):
    def body(r, carry):
        for c4 in range(width // 16):
            buf[r, pl.ds(c4 * 16, 16)] = jnp.zeros((16,), jnp.float32)
        return carry
    lax.fori_loop(0, rows, body, 0)


@functools.cache
def _build_sc_degree():
  mesh = plsc.VectorSubcoreMesh(core_axis_name="c", subcore_axis_name="s")

  @functools.partial(
      pl.kernel,
      mesh=mesh,
      out_type=jax.ShapeDtypeStruct((NW, N), jnp.float32),
      scratch_types=[
          pltpu.VMEM((EW,), jnp.int32),
          pltpu.VMEM((N,), jnp.float32),
      ],
      compiler_params=pltpu.CompilerParams(use_tc_tiling_on_sc=False,
                                           needs_layout_passes=False),
  )
  def _sc_degree(dst_hbm, out_hbm, dst_v, deg_v):
    cid = lax.axis_index("c")
    sid = lax.axis_index("s")
    wid = sid * 2 + cid

    def zb(i, carry):
        deg_v[pl.ds(i * 16, 16)] = jnp.zeros((16,), jnp.float32)
        return carry
    lax.fori_loop(0, N // 16, zb, 0)

    pltpu.sync_copy(dst_hbm.at[wid], dst_v)
    ones = jnp.ones((16,), jnp.float32)

    def body(i, carry):
        idx = dst_v[pl.ds(i * 16, 16)]
        plsc.addupdate_scatter(deg_v, [idx], ones)
        return carry
    lax.fori_loop(0, (EW // 16) * 16 // 16, body, 0)
    idx_t = dst_v[pl.ds(EW - 16, 16)]
    lanemask = lax.iota(jnp.int32, 16) >= (16 - (EW - (EW // 16) * 16) or 16)
    plsc.addupdate_scatter(deg_v, [idx_t], ones, mask=lanemask)

    pltpu.sync_copy(deg_v, out_hbm.at[wid])

  return _sc_degree


@functools.cache
def _build_sc_propagate(width, ch):
  mesh = plsc.VectorSubcoreMesh(core_axis_name="c", subcore_axis_name="s")
  nch = EW // ch

  scratch = [
      pltpu.VMEM((nch, ch), jnp.int32),
      pltpu.VMEM((nch, ch), jnp.int32),
  ]
  scratch += [pltpu.VMEM((ch, width), jnp.float32) for _ in range(NBUF)]
  scratch.append(pltpu.VMEM_SHARED((NP, width), jnp.float32))
  scratch += [pltpu.SemaphoreType.DMA for _ in range(NBUF)]

  @functools.partial(
      pl.kernel,
      mesh=mesh,
      out_type=jax.ShapeDtypeStruct((2, NP, width), jnp.float32),
      scratch_types=scratch,
      compiler_params=pltpu.CompilerParams(use_tc_tiling_on_sc=False),
  )
  def _sc_propagate(src_hbm, dst_hbm, u_hbm, out_hbm,
                    src_v, dst_v, *rest):
    bufs = rest[:NBUF]
    acc_sh = rest[NBUF]
    sems = rest[NBUF + 1:]
    cid = lax.axis_index("c")
    sid = lax.axis_index("s")
    wid = sid * 2 + cid

    zch = (ch // 8) * 8
    _zero_vmem(bufs[0], zch, width)
    nz, rem = RPT // zch, RPT % zch
    for j in range(nz):
        pltpu.sync_copy(bufs[0].at[pl.ds(0, zch)],
                        acc_sh.at[pl.ds(sid * RPT + j * zch, zch)])
    if rem:
        pltpu.sync_copy(bufs[0].at[pl.ds(0, rem)],
                        acc_sh.at[pl.ds(sid * RPT + nz * zch, rem)])
    plsc.subcore_barrier()

    pltpu.sync_copy(src_hbm.at[wid], src_v)
    pltpu.sync_copy(dst_hbm.at[wid], dst_v)
    pend = []
    for c in range(min(NBUF - 1, nch)):
        pend.append(pltpu.async_copy(u_hbm.at[src_v.at[c]],
                                     bufs[c % NBUF], sems[c % NBUF]))
    for c in range(nch):
        if c + NBUF - 1 < nch:
            nb = (c + NBUF - 1) % NBUF
            pend.append(pltpu.async_copy(u_hbm.at[src_v.at[c + NBUF - 1]],
                                         bufs[nb], sems[nb]))
        pend.pop(0).wait()
        pltpu.sync_copy(bufs[c % NBUF], acc_sh.at[dst_v.at[c]], add=True)
    plsc.subcore_barrier()

    pltpu.sync_copy(acc_sh.at[pl.ds(sid * RPT, RPT)],
                    out_hbm.at[cid, pl.ds(sid * RPT, RPT)])

  return _sc_propagate


def _k_mm(x_ref, w_ref, o_ref):
    o_ref[...] = jnp.dot(x_ref[...], w_ref[...],
                         preferred_element_type=jnp.float32)


def _k_mm_dinv_u(x_ref, w_ref, dp_ref, dinv_ref, u_ref):
    h = jnp.dot(x_ref[...], w_ref[...], preferred_element_type=jnp.float32)
    deg = 1.0 + lax.dot_general(
        dp_ref[...], jnp.ones((NW, 16), jnp.float32),
        dimension_numbers=(((0,), (0,)), ((), ())),
        preferred_element_type=jnp.float32)
    dinv = lax.rsqrt(deg)
    dinv_ref[...] = dinv
    u_ref[...] = dinv[:, 0:1] * h


def _k_relu_scale(vp_ref, u_ref, dinv_ref, b_ref, ub_ref):
    d = dinv_ref[:, 0:1]
    p = d * (vp_ref[0] + vp_ref[1] + u_ref[...])
    ub_ref[...] = d * jnp.maximum(p + b_ref[...], 0.0)


def _k_dual(vp_ref, u_ref, dinv_ref, ws_ref, bs_ref, ww_ref, bw_ref,
            u34_ref):
    d = dinv_ref[:, 0:1]
    pb = d * (vp_ref[0] + vp_ref[1] + u_ref[...])
    u34_ref[:, 0:D_HID] = d * jnp.maximum(
        jnp.dot(pb, ws_ref[...], preferred_element_type=jnp.float32)
        + bs_ref[...], 0.0)
    u34_ref[:, D_HID:2 * D_HID] = d * jnp.maximum(
        jnp.dot(pb, ww_ref[...], preferred_element_type=jnp.float32)
        + bw_ref[...], 0.0)


def _k_final2(vp_ref, u_ref, dinv_ref, ws_ref, bs_ref, ww_ref, bw_ref,
              xs_ref, xw_ref):
    d = dinv_ref[:, 0:1]
    ps = d * (vp_ref[0, :, 0:D_HID] + vp_ref[1, :, 0:D_HID]
              + u_ref[:, 0:D_HID])
    pw = d * (vp_ref[0, :, D_HID:2 * D_HID] + vp_ref[1, :, D_HID:2 * D_HID]
              + u_ref[:, D_HID:2 * D_HID])
    xs_ref[...] = jnp.maximum(
        jnp.dot(ps, ws_ref[...], preferred_element_type=jnp.float32)
        + bs_ref[...], 0.0)
    xw_ref[...] = jnp.maximum(
        jnp.dot(pw, ww_ref[...], preferred_element_type=jnp.float32)
        + bw_ref[...], 0.0)


def _row_spec(blk, width):
    return pl.BlockSpec((blk, width), lambda i: (i, 0))


def _rep_spec(shape):
    nd = len(shape)
    return pl.BlockSpec(shape, lambda i: (0,) * nd)


def _part_spec(blk, width):
    return pl.BlockSpec((2, blk, width), lambda i: (0, i, 0))


def kernel(x, edges, W_b, b_b, W_s1, b_s1, W_s2, b_s2, W_w1, b_w1, W_w2, b_w2):
    edges = edges.astype(jnp.int32)
    src3 = edges[:, 0].reshape(NW, NCH, CH)
    dst3 = edges[:, 1].reshape(NW, NCH, CH)
    dst2 = edges[:, 1].reshape(NW, EW)
    src3b = edges[:, 0].reshape(NW, EW // 100, 100)
    dst3b = edges[:, 1].reshape(NW, EW // 100, 100)
    b_b2 = b_b.reshape(1, D_HID)
    b_s12 = b_s1.reshape(1, D_HID)
    b_s22 = b_s2.reshape(1, D_IN)
    b_w12 = b_w1.reshape(1, D_HID)
    b_w22 = b_w2.reshape(1, D_IN)

    _sc_degree = _build_sc_degree()
    _sc_prop64 = _build_sc_propagate(D_HID, CH)
    _sc_prop128 = _build_sc_propagate(2 * D_HID, 100)

    degp = _sc_degree(dst2)
    dinv, u1 = pl.pallas_call(
        _k_mm_dinv_u,
        grid=(10,),
        in_specs=[_row_spec(1024, D_IN), _rep_spec((D_IN, D_HID)),
                  pl.BlockSpec((NW, 1024), lambda i: (0, i))],
        out_specs=[_row_spec(1024, 16), _row_spec(1024, D_HID)],
        out_shape=[jax.ShapeDtypeStruct((N, 16), jnp.float32),
                   jax.ShapeDtypeStruct((N, D_HID), jnp.float32)],
    )(x, W_b, degp)

    v1p = _sc_prop64(src3, dst3, u1)

    ub = pl.pallas_call(
        _k_relu_scale,
        grid=(5,),
        in_specs=[_part_spec(2000, D_HID), _row_spec(2000, D_HID),
                  _row_spec(2000, 16), _rep_spec((1, D_HID))],
        out_specs=_row_spec(2000, D_HID),
        out_shape=jax.ShapeDtypeStruct((N, D_HID), jnp.float32),
    )(v1p, u1, dinv, b_b2)

    vbp = _sc_prop64(src3, dst3, ub)

    u34 = pl.pallas_call(
        _k_dual,
        grid=(5,),
        in_specs=[_part_spec(2000, D_HID), _row_spec(2000, D_HID),
                  _row_spec(2000, 16),
                  _rep_spec((D_HID, D_HID)), _rep_spec((1, D_HID)),
                  _rep_spec((D_HID, D_HID)), _rep_spec((1, D_HID))],
        out_specs=_row_spec(2000, 2 * D_HID),
        out_shape=jax.ShapeDtypeStruct((N, 2 * D_HID), jnp.float32),
    )(vbp, ub, dinv, W_s1, b_s12, W_w1, b_w12)

    v34p = _sc_prop128(src3b, dst3b, u34)
    xs, xw = pl.pallas_call(
        _k_final2,
        grid=(10,),
        in_specs=[_part_spec(1000, 2 * D_HID), _row_spec(1000, 2 * D_HID),
                  _row_spec(1000, 16),
                  _rep_spec((D_HID, D_IN)), _rep_spec((1, D_IN)),
                  _rep_spec((D_HID, D_IN)), _rep_spec((1, D_IN))],
        out_specs=[_row_spec(1000, D_IN), _row_spec(1000, D_IN)],
        out_shape=[jax.ShapeDtypeStruct((N, D_IN), jnp.float32),
                   jax.ShapeDtypeStruct((N, D_IN), jnp.float32)],
    )(v34p, u34, dinv, W_s2, b_s22, W_w2, b_w22)

    return (xs, xw)


# speedup vs baseline: 1.1550x; 1.0073x over previous
import functools

import jax
import jax.numpy as jnp
from jax import lax
from jax.experimental import pallas as pl
from jax.experimental.pallas import tpu as pltpu
from jax.experimental.pallas import tpu_sc as plsc

N = 10000
E = 160000
D_IN = 640
D_HID = 64

NW = 32
EW = E // NW
CH = 125
NCH = EW // CH
NBUF = 3
NP = 10240
RPT = NP // 16
ZCH = 128

def _zero_vmem(buf, rows, width):
    def body(r, carry):
        for c4 in range(width // 16):
            buf[r, pl.ds(c4 * 16, 16)] = jnp.zeros((16,), jnp.float32)
        return carry
    lax.fori_loop(0, rows, body, 0)


@functools.cache
def _build_sc_degree():
  mesh = plsc.VectorSubcoreMesh(core_axis_name="c", subcore_axis_name="s")

  @functools.partial(
      pl.kernel,
      mesh=mesh,
      out_type=jax.ShapeDtypeStruct((NW, N), jnp.float32),
      scratch_types=[
          pltpu.VMEM((EW,), jnp.int32),
          pltpu.VMEM((N,), jnp.float32),
      ],
      compiler_params=pltpu.CompilerParams(use_tc_tiling_on_sc=False,
                                           needs_layout_passes=False),
  )
  def _sc_degree(dst_hbm, out_hbm, dst_v, deg_v):
    cid = lax.axis_index("c")
    sid = lax.axis_index("s")
    wid = sid * 2 + cid

    def zb(i, carry):
        deg_v[pl.ds(i * 16, 16)] = jnp.zeros((16,), jnp.float32)
        return carry
    lax.fori_loop(0, N // 16, zb, 0)

    pltpu.sync_copy(dst_hbm.at[wid], dst_v)
    ones = jnp.ones((16,), jnp.float32)

    def body(i, carry):
        idx = dst_v[pl.ds(i * 16, 16)]
        plsc.addupdate_scatter(deg_v, [idx], ones)
        return carry
    lax.fori_loop(0, (EW // 16) * 16 // 16, body, 0)
    idx_t = dst_v[pl.ds(EW - 16, 16)]
    lanemask = lax.iota(jnp.int32, 16) >= (16 - (EW - (EW // 16) * 16) or 16)
    plsc.addupdate_scatter(deg_v, [idx_t], ones, mask=lanemask)

    pltpu.sync_copy(deg_v, out_hbm.at[wid])

  return _sc_degree


@functools.cache
def _build_sc_propagate(width, ch, nbuf=NBUF):
  mesh = plsc.VectorSubcoreMesh(core_axis_name="c", subcore_axis_name="s")
  nch = EW // ch

  scratch = [
      pltpu.VMEM((nch, ch), jnp.int32),
      pltpu.VMEM((nch, ch), jnp.int32),
  ]
  scratch += [pltpu.VMEM((ch, width), jnp.float32) for _ in range(nbuf)]
  scratch.append(pltpu.VMEM_SHARED((NP, width), jnp.float32))
  scratch += [pltpu.SemaphoreType.DMA for _ in range(nbuf)]

  @functools.partial(
      pl.kernel,
      mesh=mesh,
      out_type=jax.ShapeDtypeStruct((2, NP, width), jnp.float32),
      scratch_types=scratch,
      compiler_params=pltpu.CompilerParams(use_tc_tiling_on_sc=False),
  )
  def _sc_propagate(src_hbm, dst_hbm, u_hbm, out_hbm,
                    src_v, dst_v, *rest):
    bufs = rest[:nbuf]
    acc_sh = rest[nbuf]
    sems = rest[nbuf + 1:]
    cid = lax.axis_index("c")
    sid = lax.axis_index("s")
    wid = sid * 2 + cid

    zch = (ch // 8) * 8
    _zero_vmem(bufs[0], zch, width)
    nz, rem = RPT // zch, RPT % zch
    for j in range(nz):
        pltpu.sync_copy(bufs[0].at[pl.ds(0, zch)],
                        acc_sh.at[pl.ds(sid * RPT + j * zch, zch)])
    if rem:
        pltpu.sync_copy(bufs[0].at[pl.ds(0, rem)],
                        acc_sh.at[pl.ds(sid * RPT + nz * zch, rem)])
    plsc.subcore_barrier()

    pltpu.sync_copy(src_hbm.at[wid], src_v)
    pltpu.sync_copy(dst_hbm.at[wid], dst_v)
    pend = []
    for c in range(min(nbuf - 1, nch)):
        pend.append(pltpu.async_copy(u_hbm.at[src_v.at[c]],
                                     bufs[c % nbuf], sems[c % nbuf]))
    for c in range(nch):
        if c + nbuf - 1 < nch:
            nb = (c + nbuf - 1) % nbuf
            pend.append(pltpu.async_copy(u_hbm.at[src_v.at[c + nbuf - 1]],
                                         bufs[nb], sems[nb]))
        pend.pop(0).wait()
        pltpu.sync_copy(bufs[c % nbuf], acc_sh.at[dst_v.at[c]], add=True)
    plsc.subcore_barrier()

    pltpu.sync_copy(acc_sh.at[pl.ds(sid * RPT, RPT)],
                    out_hbm.at[cid, pl.ds(sid * RPT, RPT)])

  return _sc_propagate


def _k_mm(x_ref, w_ref, o_ref):
    o_ref[...] = jnp.dot(x_ref[...], w_ref[...],
                         preferred_element_type=jnp.float32)


def _k_mm_dinv_u(x_ref, w_ref, dp_ref, dinv_ref, u_ref):
    h = jnp.dot(x_ref[...], w_ref[...], preferred_element_type=jnp.float32)
    deg = 1.0 + lax.dot_general(
        dp_ref[...], jnp.ones((NW, 16), jnp.float32),
        dimension_numbers=(((0,), (0,)), ((), ())),
        preferred_element_type=jnp.float32)
    dinv = lax.rsqrt(deg)
    dinv_ref[...] = dinv
    u_ref[...] = dinv[:, 0:1] * h


def _k_relu_scale(vp_ref, u_ref, dinv_ref, b_ref, ub_ref):
    d = dinv_ref[:, 0:1]
    p = d * (vp_ref[0] + vp_ref[1] + u_ref[...])
    ub_ref[...] = d * jnp.maximum(p + b_ref[...], 0.0)


def _k_dual(vp_ref, u_ref, dinv_ref, ws_ref, bs_ref, ww_ref, bw_ref,
            u34_ref):
    d = dinv_ref[:, 0:1]
    pb = d * (vp_ref[0] + vp_ref[1] + u_ref[...])
    u34_ref[:, 0:D_HID] = d * jnp.maximum(
        jnp.dot(pb, ws_ref[...], preferred_element_type=jnp.float32)
        + bs_ref[...], 0.0)
    u34_ref[:, D_HID:2 * D_HID] = d * jnp.maximum(
        jnp.dot(pb, ww_ref[...], preferred_element_type=jnp.float32)
        + bw_ref[...], 0.0)


def _k_final2(vp_ref, u_ref, dinv_ref, ws_ref, bs_ref, ww_ref, bw_ref,
              xs_ref, xw_ref):
    d = dinv_ref[:, 0:1]
    ps = d * (vp_ref[0, :, 0:D_HID] + vp_ref[1, :, 0:D_HID]
              + u_ref[:, 0:D_HID])
    pw = d * (vp_ref[0, :, D_HID:2 * D_HID] + vp_ref[1, :, D_HID:2 * D_HID]
              + u_ref[:, D_HID:2 * D_HID])
    xs_ref[...] = jnp.maximum(
        jnp.dot(ps, ws_ref[...], preferred_element_type=jnp.float32)
        + bs_ref[...], 0.0)
    xw_ref[...] = jnp.maximum(
        jnp.dot(pw, ww_ref[...], preferred_element_type=jnp.float32)
        + bw_ref[...], 0.0)


def _row_spec(blk, width):
    return pl.BlockSpec((blk, width), lambda i: (i, 0))


def _rep_spec(shape):
    nd = len(shape)
    return pl.BlockSpec(shape, lambda i: (0,) * nd)


def _part_spec(blk, width):
    return pl.BlockSpec((2, blk, width), lambda i: (0, i, 0))


def kernel(x, edges, W_b, b_b, W_s1, b_s1, W_s2, b_s2, W_w1, b_w1, W_w2, b_w2):
    edges = edges.astype(jnp.int32)
    src3 = edges[:, 0].reshape(NW, NCH, CH)
    dst3 = edges[:, 1].reshape(NW, NCH, CH)
    dst2 = edges[:, 1].reshape(NW, EW)
    src3b = edges[:, 0].reshape(NW, EW // 50, 50)
    dst3b = edges[:, 1].reshape(NW, EW // 50, 50)
    b_b2 = b_b.reshape(1, D_HID)
    b_s12 = b_s1.reshape(1, D_HID)
    b_s22 = b_s2.reshape(1, D_IN)
    b_w12 = b_w1.reshape(1, D_HID)
    b_w22 = b_w2.reshape(1, D_IN)

    _sc_degree = _build_sc_degree()
    _sc_prop64 = _build_sc_propagate(D_HID, CH, 4)
    _sc_prop128 = _build_sc_propagate(2 * D_HID, 50, 4)

    degp = _sc_degree(dst2)
    dinv, u1 = pl.pallas_call(
        _k_mm_dinv_u,
        grid=(10,),
        in_specs=[_row_spec(1024, D_IN), _rep_spec((D_IN, D_HID)),
                  pl.BlockSpec((NW, 1024), lambda i: (0, i))],
        out_specs=[_row_spec(1024, 16), _row_spec(1024, D_HID)],
        out_shape=[jax.ShapeDtypeStruct((N, 16), jnp.float32),
                   jax.ShapeDtypeStruct((N, D_HID), jnp.float32)],
    )(x, W_b, degp)

    v1p = _sc_prop64(src3, dst3, u1)

    ub = pl.pallas_call(
        _k_relu_scale,
        grid=(5,),
        in_specs=[_part_spec(2000, D_HID), _row_spec(2000, D_HID),
                  _row_spec(2000, 16), _rep_spec((1, D_HID))],
        out_specs=_row_spec(2000, D_HID),
        out_shape=jax.ShapeDtypeStruct((N, D_HID), jnp.float32),
    )(v1p, u1, dinv, b_b2)

    vbp = _sc_prop64(src3, dst3, ub)

    u34 = pl.pallas_call(
        _k_dual,
        grid=(5,),
        in_specs=[_part_spec(2000, D_HID), _row_spec(2000, D_HID),
                  _row_spec(2000, 16),
                  _rep_spec((D_HID, D_HID)), _rep_spec((1, D_HID)),
                  _rep_spec((D_HID, D_HID)), _rep_spec((1, D_HID))],
        out_specs=_row_spec(2000, 2 * D_HID),
        out_shape=jax.ShapeDtypeStruct((N, 2 * D_HID), jnp.float32),
    )(vbp, ub, dinv, W_s1, b_s12, W_w1, b_w12)

    v34p = _sc_prop128(src3b, dst3b, u34)
    xs, xw = pl.pallas_call(
        _k_final2,
        grid=(10,),
        in_specs=[_part_spec(1000, 2 * D_HID), _row_spec(1000, 2 * D_HID),
                  _row_spec(1000, 16),
                  _rep_spec((D_HID, D_IN)), _rep_spec((1, D_IN)),
                  _rep_spec((D_HID, D_IN)), _rep_spec((1, D_IN))],
        out_specs=[_row_spec(1000, D_IN), _row_spec(1000, D_IN)],
        out_shape=[jax.ShapeDtypeStruct((N, D_IN), jnp.float32),
                   jax.ShapeDtypeStruct((N, D_IN), jnp.float32)],
    )(v34p, u34, dinv, W_s2, b_s22, W_w2, b_w22)

    return (xs, xw)
